# baseline (speedup 1.0000x reference)
import functools

import jax
import jax.numpy as jnp
from jax import lax
from jax.experimental import pallas as pl
from jax.experimental.pallas import tpu as pltpu
from jax.experimental.pallas import tpu_sc as plsc

_B, _S, _N = 8, 1024, 32
_HID = 128
_L = 16
_NC, _NS = 2, 16
_NW = _NC * _NS
_SLOTS = _B * _S * _N // _NW
_ROWS = _B * _S // _NW
_CHI = 128
_CH = _SLOTS // _CHI
_TROWS = 9 * 1024
_ROW_IDX0 = 8192
_ROW_MASKED = 8193


def _build_table(h, ptail, pt0, Wn):
    def body(h_ref, ptail_ref, pt0_ref, wn_ref, out_ref):
        i = pl.program_id(0)
        ws = wn_ref[...] * (1.0 / _N)

        @pl.when(i < _B)
        def _():
            t = h_ref[0] + ptail_ref[...]
            out_ref[...] = lax.dot_general(
                t, ws, (((1,), (1,)), ((), ())),
                preferred_element_type=jnp.float32)

        @pl.when(i == _B)
        def _():
            rows = lax.broadcasted_iota(jnp.int32, (_S, _HID), 0)
            blk = jnp.where(rows == 0, pt0_ref[...], 0.0)
            out_ref[...] = lax.dot_general(
                blk, ws, (((1,), (1,)), ((), ())),
                preferred_element_type=jnp.float32)

    return pl.pallas_call(
        body,
        grid=(_B + 1,),
        in_specs=[
            pl.BlockSpec((1, _S, _HID), lambda i: (jnp.minimum(i, _B - 1), 0, 0)),
            pl.BlockSpec((_S, _HID), lambda i: (0, 0)),
            pl.BlockSpec((1, _HID), lambda i: (0, 0)),
            pl.BlockSpec((_HID, _HID), lambda i: (0, 0)),
        ],
        out_specs=pl.BlockSpec((_S, _HID), lambda i: (i, 0)),
        out_shape=jax.ShapeDtypeStruct((_TROWS, _HID), jnp.float32),
    )(h, ptail, pt0, Wn)


@functools.lru_cache(maxsize=1)
def _make_sc_gather():
    mesh = plsc.VectorSubcoreMesh(core_axis_name="c", subcore_axis_name="s")

    @functools.partial(
        pl.kernel,
        mesh=mesh,
        out_type=jax.ShapeDtypeStruct((_B * _S, _HID), jnp.float32),
        scratch_types=[
            pltpu.VMEM((_SLOTS,), jnp.int32),
            pltpu.VMEM((_SLOTS,), jnp.int32),
            pltpu.VMEM((_CH, _CHI), jnp.int32),
            pltpu.VMEM((2, _CHI, _HID), jnp.float32),
            pltpu.VMEM((_ROWS, _HID), jnp.float32),
            pltpu.SemaphoreType.DMA,
            pltpu.SemaphoreType.DMA,
        ],
    )
    def sc_gather(t_hbm, idx_hbm, msk_hbm, out_hbm,
                  idx_v, msk_v, gidx_v, rbuf, oacc, sem0, sem1):
        wid = lax.axis_index("s") * _NC + lax.axis_index("c")
        base = wid * _SLOTS
        rbase = wid * _ROWS
        boff = wid // (_NW // _B) * _S - 1

        pltpu.sync_copy(idx_hbm.at[pl.ds(base, _SLOTS)], idx_v)
        pltpu.sync_copy(msk_hbm.at[pl.ds(base, _SLOTS)], msk_v)

        def gix_body(ch, _):
            for c in range(_CHI // _L):
                off = ch * _CHI + c * _L
                m = msk_v[pl.ds(off, _L)]
                v = idx_v[pl.ds(off, _L)]
                g = jnp.where(m != 0,
                              jnp.where(v == 0, _ROW_IDX0, v + boff),
                              _ROW_MASKED)
                gidx_v[ch, pl.ds(c * _L, _L)] = g
            return 0

        lax.fori_loop(0, _CH, gix_body, 0)

        sems = (sem0, sem1)
        pltpu.async_copy(t_hbm.at[gidx_v.at[0]], rbuf.at[0], sems[0])

        def pair_body(p, _):
            for k in range(2):
                ch = 2 * p + k
                nxt = (k + 1) % 2

                @pl.when(ch + 1 < _CH)
                def _():
                    pltpu.async_copy(
                        t_hbm.at[gidx_v.at[ch + 1]], rbuf.at[nxt], sems[nxt])

                pltpu.make_async_copy(
                    t_hbm.at[gidx_v.at[ch]], rbuf.at[k], sems[k]).wait()

                for j in range(_CHI // _N):
                    accs = [rbuf[k, j * _N, pl.ds(c * _L, _L)]
                            for c in range(_HID // _L)]
                    for n in range(1, _N):
                        for c in range(_HID // _L):
                            accs[c] = accs[c] + rbuf[k, j * _N + n,
                                                     pl.ds(c * _L, _L)]
                    row = ch * (_CHI // _N) + j
                    for c in range(_HID // _L):
                        oacc[row, pl.ds(c * _L, _L)] = accs[c]
            return 0

        lax.fori_loop(0, _CH // 2, pair_body, 0)
        pltpu.sync_copy(oacc, out_hbm.at[pl.ds(rbase, _ROWS)])

    return sc_gather


def kernel(x, h, g, neighbor_index, neighbor_mask, Wn, pos_table):
    del x, g
    table = _build_table(h, pos_table[1:], pos_table[0:1], Wn)
    idx_flat = neighbor_index.astype(jnp.int32).reshape(-1)
    msk_flat = neighbor_mask.astype(jnp.int32).reshape(-1)
    acc = _make_sc_gather()(table, idx_flat, msk_flat)
    return acc.reshape(_B, _S, _HID)


# speedup vs baseline: 40.3231x; 40.3231x over previous
import functools

import jax
import jax.numpy as jnp
from jax import lax
from jax.experimental import pallas as pl
from jax.experimental.pallas import tpu as pltpu
from jax.experimental.pallas import tpu_sc as plsc

_B, _S, _N = 8, 1024, 32
_HID = 128
_L = 16
_NC, _NS = 2, 16
_NW = _NC * _NS
_NRG = 16
_HCOL = _HID // 2
_GROWS = _B * _S // _NRG
_GSLOTS = _GROWS * _N
_TROWS = 9 * 1024
_LT_ROWS = _S + 2
_MCH = _GSLOTS // 2

_GDN = lax.GatherDimensionNumbers(
    offset_dims=(), collapsed_slice_dims=(0,), start_index_map=(0,))


def _take16(vec, idx):
    return lax.gather(vec, idx[:, None], dimension_numbers=_GDN,
                      slice_sizes=(1,),
                      mode=lax.GatherScatterMode.PROMISE_IN_BOUNDS)


def _build_table(h, ptail, pt0, Wn):
    def body(h_ref, ptail_ref, pt0_ref, wn_ref, out_ref):
        i = pl.program_id(0)
        ws = wn_ref[...] * (1.0 / _N)

        @pl.when(i < _B)
        def _():
            t = h_ref[0] + ptail_ref[...]
            out_ref[0] = lax.dot_general(
                t, ws, (((1,), (1,)), ((), ())),
                preferred_element_type=jnp.float32)

        @pl.when(i == _B)
        def _():
            rows = lax.broadcasted_iota(jnp.int32, (_S, _HID), 0)
            blk = jnp.where(rows == 0, pt0_ref[...], 0.0)
            out_ref[0] = lax.dot_general(
                blk, ws, (((1,), (1,)), ((), ())),
                preferred_element_type=jnp.float32)

    return pl.pallas_call(
        body,
        grid=(_B + 1, 2),
        in_specs=[
            pl.BlockSpec((1, _S, _HID),
                         lambda i, half: (jnp.minimum(i, _B - 1), 0, 0)),
            pl.BlockSpec((_S, _HID), lambda i, half: (0, 0)),
            pl.BlockSpec((1, _HID), lambda i, half: (0, 0)),
            pl.BlockSpec((_HCOL, _HID), lambda i, half: (half, 0)),
        ],
        out_specs=pl.BlockSpec((1, _S, _HCOL), lambda i, half: (half, i, 0)),
        out_shape=jax.ShapeDtypeStruct((2, _TROWS, _HCOL), jnp.float32),
    )(h, ptail, pt0, Wn)


@functools.lru_cache(maxsize=1)
def _make_sc_gather():
    mesh = plsc.VectorSubcoreMesh(core_axis_name="c", subcore_axis_name="s")

    @functools.partial(
        pl.kernel,
        mesh=mesh,
        compiler_params=pltpu.CompilerParams(
            needs_layout_passes=False, use_tc_tiling_on_sc=False),
        out_type=jax.ShapeDtypeStruct((2, _B * _S, _HCOL), jnp.float32),
        scratch_types=[
            pltpu.VMEM((_LT_ROWS, _HCOL), jnp.float32),
            pltpu.VMEM((_GSLOTS,), jnp.int32),
            pltpu.VMEM((_MCH,), jnp.int32),
            pltpu.VMEM((_GROWS, _HCOL), jnp.float32),
        ],
    )
    def sc_gather(t_hbm, idx_hbm, msk_hbm, out_hbm, ttile, idx_v, msk_v, oacc):
        wid = lax.axis_index("s") * _NC + lax.axis_index("c")
        rg = wid % _NRG
        half = wid // _NRG
        b = rg // (_NRG // _B)
        sbase = rg * _GSLOTS

        pltpu.sync_copy(t_hbm.at[half, pl.ds(b * _S, _S)],
                        ttile.at[pl.ds(0, _S)])
        pltpu.sync_copy(t_hbm.at[half, pl.ds(_B * _S, 2)],
                        ttile.at[pl.ds(_S, 2)])
        pltpu.sync_copy(idx_hbm.at[pl.ds(sbase, _GSLOTS)], idx_v)

        for p in range(2):
            pltpu.sync_copy(msk_hbm.at[pl.ds(sbase + p * _MCH, _MCH)], msk_v)

            def remap_body(i, _):
                off = i * _L
                m = msk_v[pl.ds(off, _L)]
                v = idx_v[pl.ds(p * _MCH + off, _L)]
                g = jnp.where(m != 0,
                              jnp.where(v == 0, _S, v - 1),
                              _S + 1)
                idx_v[pl.ds(p * _MCH + off, _L)] = g
                return 0

            lax.fori_loop(0, _MCH // _L, remap_body, 0)

        coffs = [jnp.arange(_L, dtype=jnp.int32) + c * _L
                 for c in range(_HCOL // _L)]
        lane_consts = [jnp.full((_L,), n, dtype=jnp.int32) for n in range(_L)]
        zero = jnp.zeros((_L,), jnp.float32)

        def row_body(s, _):
            accs = [zero for _ in range(_HCOL // _L)]
            for hblk in range(2):
                iv = idx_v[pl.ds(s * _N + hblk * _L, _L)]
                for n in range(_L):
                    spl = _take16(iv, lane_consts[n])
                    for c in range(_HCOL // _L):
                        val = plsc.load_gather(ttile, [spl, coffs[c]])
                        accs[c] = accs[c] + val
            for c in range(_HCOL // _L):
                oacc[s, pl.ds(c * _L, _L)] = accs[c]
            return 0

        lax.fori_loop(0, _GROWS, row_body, 0)
        pltpu.sync_copy(oacc, out_hbm.at[half, pl.ds(rg * _GROWS, _GROWS)])

    return sc_gather


def kernel(x, h, g, neighbor_index, neighbor_mask, Wn, pos_table):
    del x, g
    table = _build_table(h, pos_table[1:], pos_table[0:1], Wn)
    idx_flat = neighbor_index.astype(jnp.int32).reshape(-1)
    msk_flat = neighbor_mask.astype(jnp.int32).reshape(-1)
    halves = _make_sc_gather()(table, idx_flat, msk_flat)
    return jnp.concatenate([halves[0], halves[1]], axis=-1).reshape(
        _B, _S, _HID)


# speedup vs baseline: 40.8326x; 1.0126x over previous
import functools

import jax
import jax.numpy as jnp
from jax import lax
from jax.experimental import pallas as pl
from jax.experimental.pallas import tpu as pltpu
from jax.experimental.pallas import tpu_sc as plsc

_B, _S, _N = 8, 1024, 32
_HID = 128
_L = 16
_NC, _NS = 2, 16
_NW = _NC * _NS
_NRG = 16
_HCOL = _HID // 2
_GROWS = _B * _S // _NRG
_GSLOTS = _GROWS * _N
_TROWS = 9 * 1024
_LT_ROWS = _S + 2
_MCH = _GSLOTS // 2

_GDN = lax.GatherDimensionNumbers(
    offset_dims=(), collapsed_slice_dims=(0,), start_index_map=(0,))


def _take16(vec, idx):
    return lax.gather(vec, idx[:, None], dimension_numbers=_GDN,
                      slice_sizes=(1,),
                      mode=lax.GatherScatterMode.PROMISE_IN_BOUNDS)


def _build_table(h, ptail, pt0, Wn):
    def body(h_ref, ptail_ref, pt0_ref, wn_ref, out_ref):
        ws = wn_ref[...] * (1.0 / _N)
        t = (h_ref[...] + ptail_ref[...][None]).reshape(_B * _S, _HID)
        rows = lax.broadcasted_iota(jnp.int32, (_S, _HID), 0)
        special = jnp.where(rows == 0, pt0_ref[...], 0.0)
        full = jnp.concatenate([t, special], axis=0)
        res = lax.dot_general(full, ws, (((1,), (1,)), ((), ())),
                              preferred_element_type=jnp.float32)
        out_ref[0] = res[:, :_HCOL]
        out_ref[1] = res[:, _HCOL:]

    return pl.pallas_call(
        body,
        out_shape=jax.ShapeDtypeStruct((2, _TROWS, _HCOL), jnp.float32),
    )(h, ptail, pt0, Wn)


@functools.lru_cache(maxsize=1)
def _make_sc_gather():
    mesh = plsc.VectorSubcoreMesh(core_axis_name="c", subcore_axis_name="s")

    @functools.partial(
        pl.kernel,
        mesh=mesh,
        compiler_params=pltpu.CompilerParams(
            needs_layout_passes=False, use_tc_tiling_on_sc=False),
        out_type=jax.ShapeDtypeStruct((_B * _S, _HID), jnp.float32),
        scratch_types=[
            pltpu.VMEM((_LT_ROWS * _HCOL,), jnp.float32),
            pltpu.VMEM((_GSLOTS,), jnp.int32),
            pltpu.VMEM((_MCH,), jnp.int32),
            pltpu.VMEM((_GROWS, _HCOL), jnp.float32),
            pltpu.SemaphoreType.DMA,
            pltpu.SemaphoreType.DMA,
            pltpu.SemaphoreType.DMA,
        ],
    )
    def sc_gather(t_hbm, idx_hbm, msk_hbm, out_hbm,
                  ttile, idx_v, msk_v, oacc, sem_t, sem_i, sem_m):
        wid = lax.axis_index("s") * _NC + lax.axis_index("c")
        rg = wid % _NRG
        half = wid // _NRG
        b = rg // (_NRG // _B)
        sbase = rg * _GSLOTS

        pltpu.async_copy(t_hbm.at[half, pl.ds(b * _S * _HCOL, _S * _HCOL)],
                         ttile.at[pl.ds(0, _S * _HCOL)], sem_t)
        pltpu.async_copy(
            t_hbm.at[half, pl.ds(_B * _S * _HCOL, 2 * _HCOL)],
            ttile.at[pl.ds(_S * _HCOL, 2 * _HCOL)], sem_t)
        pltpu.async_copy(idx_hbm.at[pl.ds(sbase, _GSLOTS)], idx_v, sem_i)
        pltpu.async_copy(msk_hbm.at[pl.ds(sbase, _MCH)], msk_v, sem_m)
        pltpu.make_async_copy(idx_hbm.at[pl.ds(sbase, _GSLOTS)], idx_v,
                              sem_i).wait()

        for p in range(2):
            pltpu.make_async_copy(msk_hbm.at[pl.ds(sbase, _MCH)], msk_v,
                                  sem_m).wait()

            def remap_body(i, _):
                off = i * _L
                m = msk_v[pl.ds(off, _L)]
                v = idx_v[pl.ds(p * _MCH + off, _L)]
                g = jnp.where(m != 0,
                              jnp.where(v == 0, _S, v - 1),
                              _S + 1)
                idx_v[pl.ds(p * _MCH + off, _L)] = g * _HCOL
                return 0

            if p == 0:
                lax.fori_loop(0, _MCH // _L, remap_body, 0)
                pltpu.async_copy(msk_hbm.at[pl.ds(sbase + _MCH, _MCH)],
                                 msk_v, sem_m)
            else:
                lax.fori_loop(0, _MCH // _L, remap_body, 0)

        coffs = [jnp.arange(_L, dtype=jnp.int32) + c * _L
                 for c in range(_HCOL // _L)]
        lane_consts = [jnp.full((_L,), n, dtype=jnp.int32) for n in range(_L)]
        zero = jnp.zeros((_L,), jnp.float32)

        pltpu.make_async_copy(
            t_hbm.at[half, pl.ds(0, _LT_ROWS * _HCOL)], ttile, sem_t).wait()

        def row_body(s, _):
            accs = [[zero] * 4 for _ in range(_HCOL // _L)]
            for hblk in range(2):
                iv = idx_v[pl.ds(s * _N + hblk * _L, _L)]
                for n in range(_L):
                    spl = _take16(iv, lane_consts[n])
                    w = n % 4
                    for c in range(_HCOL // _L):
                        val = plsc.load_gather(ttile, [spl + coffs[c]])
                        accs[c][w] = accs[c][w] + val
            for c in range(_HCOL // _L):
                a = accs[c]
                oacc[s, pl.ds(c * _L, _L)] = (a[0] + a[1]) + (a[2] + a[3])
            return 0

        lax.fori_loop(0, _GROWS, row_body, 0)
        pltpu.sync_copy(
            oacc,
            out_hbm.at[pl.ds(rg * _GROWS, _GROWS),
                       pl.ds(half * _HCOL, _HCOL)])

    return sc_gather


def kernel(x, h, g, neighbor_index, neighbor_mask, Wn, pos_table):
    del x, g
    table = _build_table(h, pos_table[1:], pos_table[0:1], Wn)
    table_flat = table.reshape(2, _TROWS * _HCOL)
    idx_flat = neighbor_index.astype(jnp.int32).reshape(-1)
    msk_flat = neighbor_mask.astype(jnp.int32).reshape(-1)
    out = _make_sc_gather()(table_flat, idx_flat, msk_flat)
    return out.reshape(_B, _S, _HID)
